# Initial kernel scaffold; baseline (speedup 1.0000x reference)
import jax
import jax.numpy as jnp
from jax.experimental import pallas as pl


def kernel(x, edge_index, W_ih, W_hh, b_ih, b_hh, W_att, W_fc, b_fc, W_g1, a_src1, a_dst1, b_g1, W_g2, a_src2, a_dst2, b_g2, W_lin, b_lin):
    raise NotImplementedError("write your pallas kernel here")


# speedup vs baseline: 13.4832x; 13.4832x over previous
import jax
import jax.numpy as jnp
from jax import lax
from jax.experimental import pallas as pl
from jax.experimental.pallas import tpu as pltpu
from jax.experimental.pallas import tpu_sc as plsc

LAG = 10
IN_DIM = 12
HID = 64
OUT_CH = 32
NUM_CLASSES = 3
N = 100000
E = 1600000

NC = 2
NS = 16
LANES = 16

TW = 32
AW = 16
BLK = 128
SLAB = 4
NBLK = 800
NSLAB = NBLK // SLAB
EP = NC * 0 + NS * NBLK * BLK
ZROWS = 80
ACC_ROWS = 101120
ADN = N + LANES
WROWS = 6256
OUT_ROWS = NS * WROWS
BN = 2000
GRID = N // BN

_f32 = jnp.float32


def _dense_a_body(x_ref, wih_ref, whh_ref, b_ref, watt_ref, wfc_ref, bfc_ref,
                  wg1_ref, asrc_ref, adst_ref, ta_ref, tb_ref, hs_ref):
    xb = x_ref[...]
    wih = wih_ref[...]
    whh = whh_ref[...]
    bias = b_ref[...]
    h = jnp.zeros((BN, HID), _f32)
    c = jnp.zeros((BN, HID), _f32)
    for t in range(LAG):
        xt = xb[:, t * IN_DIM:(t + 1) * IN_DIM]
        gates = (jnp.dot(xt, wih, preferred_element_type=_f32)
                 + jnp.dot(h, whh, preferred_element_type=_f32) + bias)
        ig = jax.nn.sigmoid(gates[:, 0:HID])
        fg = jax.nn.sigmoid(gates[:, HID:2 * HID])
        gg = jnp.tanh(gates[:, 2 * HID:3 * HID])
        og = jax.nn.sigmoid(gates[:, 3 * HID:4 * HID])
        c = fg * c + ig * gg
        h = og * jnp.tanh(c)
        hs_ref[:, t * HID:(t + 1) * HID] = h
    hs = hs_ref[...]
    watt = watt_ref[...]
    cols = []
    for t in range(LAG):
        ht = hs[:, t * HID:(t + 1) * HID]
        cols.append(jnp.sum(ht * watt[t:t + 1, :], axis=1, keepdims=True))
    sc = jnp.concatenate(cols, axis=1)
    m = jnp.max(sc, axis=1, keepdims=True)
    ex = jnp.exp(sc - m)
    att = ex / jnp.sum(ex, axis=1, keepdims=True)
    att_ht = jnp.zeros((BN, HID), _f32)
    for t in range(LAG):
        att_ht = att_ht + att[:, t:t + 1] * hs[:, t * HID:(t + 1) * HID]
    hfc = jnp.maximum(
        jnp.dot(att_ht, wfc_ref[...], preferred_element_type=_f32)
        + bfc_ref[...], 0.0)
    g1 = jnp.dot(hfc, wg1_ref[...], preferred_element_type=_f32)
    as1 = jnp.sum(g1 * asrc_ref[...], axis=1, keepdims=True)
    ad1 = jnp.sum(g1 * adst_ref[...], axis=1, keepdims=True)
    padz = jnp.zeros((BN, TW - 18), _f32)
    ta_ref[...] = jnp.concatenate([g1[:, :16], as1, ad1, padz], axis=1)
    tb_ref[...] = jnp.concatenate([g1[:, 16:32], as1, ad1, padz], axis=1)


def _dense_a(x2d, wih, whh, b, watt, wfc, bfc, wg1, asrc, adst):
    full = lambda shp: pl.BlockSpec(shp, lambda i: (0, 0))
    return pl.pallas_call(
        _dense_a_body,
        grid=(GRID,),
        in_specs=[
            pl.BlockSpec((BN, LAG * IN_DIM), lambda i: (i, 0)),
            full((IN_DIM, 4 * HID)),
            full((HID, 4 * HID)),
            full((1, 4 * HID)),
            full((LAG, HID)),
            full((HID, HID)),
            full((1, HID)),
            full((HID, OUT_CH)),
            full((1, OUT_CH)),
            full((1, OUT_CH)),
        ],
        out_specs=[
            pl.BlockSpec((BN, TW), lambda i: (i, 0)),
            pl.BlockSpec((BN, TW), lambda i: (i, 0)),
        ],
        out_shape=[
            jax.ShapeDtypeStruct((N, TW), _f32),
            jax.ShapeDtypeStruct((N, TW), _f32),
        ],
        scratch_shapes=[pltpu.VMEM((BN, LAG * HID), _f32)],
    )(x2d, wih, whh, b, watt, wfc, bfc, wg1, asrc, adst)


def _edge_body(ta_hbm, tb_hbm, ad_hbm, src_hbm, dst_hbm,
               outa_hbm, outb_hbm, den_hbm,
               acc, den_acc, ad_s, rows, out_v, ee_v, adv, zrow, zden,
               sidx, didx, sem):
    core = lax.axis_index("c")
    tile = lax.axis_index("s")

    zeros16 = jnp.zeros((LANES,), _f32)

    def _zr(r, _):
        zrow[r, 0:LANES] = zeros16
        return 0

    lax.fori_loop(0, ZROWS, _zr, 0)
    for g in range(ZROWS // LANES):
        zden[pl.ds(g * LANES, LANES)] = zeros16

    rows_per_tile = ACC_ROWS // NS
    zbase = tile * rows_per_tile

    def _zacc(z, _):
        pltpu.sync_copy(zrow, acc.at[pl.ds(zbase + z * ZROWS, ZROWS)])
        pltpu.sync_copy(zden, den_acc.at[pl.ds(zbase + z * ZROWS, ZROWS)])
        return 0

    lax.fori_loop(0, rows_per_tile // ZROWS, _zacc, 0)

    @pl.when(tile == 0)
    def _stage_ad():
        pltpu.sync_copy(ad_hbm, ad_s)

    plsc.subcore_barrier()

    cols16 = {}

    def _c16(c):
        if c not in cols16:
            cols16[c] = jnp.full((LANES,), c, jnp.int32)
        return cols16[c]

    def _block(j):
        def _ga():
            pltpu.async_copy(ta_hbm.at[sidx.at[j]], rows, sem).wait()

        def _gb():
            pltpu.async_copy(tb_hbm.at[sidx.at[j]], rows, sem).wait()

        pl.when(core == 0)(_ga)
        pl.when(core == 1)(_gb)
        pltpu.sync_copy(ad_s.at[didx.at[j]], adv)

        for g in range(BLK // LANES):
            eids = lax.iota(jnp.int32, LANES) + (g * LANES)
            as16 = plsc.load_gather(rows, [eids, _c16(16)])
            ad16 = adv[pl.ds(g * LANES, LANES)]
            e = as16 + ad16
            ee = jnp.exp(jnp.maximum(e, 0.2 * e))
            ee_v[pl.ds(g * LANES, LANES)] = ee
            for col in range(AW):
                v = plsc.load_gather(rows, [eids, _c16(col)])
                plsc.store_scatter(out_v, [eids, _c16(col)], v * ee)
        pltpu.sync_copy(out_v, acc.at[didx.at[j]], add=True)
        pl.when(core == 0)(
            lambda: pltpu.sync_copy(ee_v, den_acc.at[didx.at[j]], add=True))

    def _slab(sl, _):
        srow = tile * NBLK + sl * SLAB
        pltpu.sync_copy(src_hbm.at[pl.ds(srow, SLAB)], sidx)
        pltpu.sync_copy(dst_hbm.at[pl.ds(srow, SLAB)], didx)
        for j in range(SLAB):
            _block(j)
        return 0

    lax.fori_loop(0, NSLAB, _slab, 0)
    plsc.subcore_barrier()

    wbase = tile * WROWS
    pl.when(core == 0)(lambda: pltpu.sync_copy(
        acc.at[pl.ds(wbase, WROWS)], outa_hbm.at[pl.ds(wbase, WROWS)]))
    pl.when(core == 0)(lambda: pltpu.sync_copy(
        den_acc.at[pl.ds(wbase, WROWS)], den_hbm.at[pl.ds(wbase, WROWS)]))
    pl.when(core == 1)(lambda: pltpu.sync_copy(
        acc.at[pl.ds(wbase, WROWS)], outb_hbm.at[pl.ds(wbase, WROWS)]))


def _edge_pass(ta, tb, ad, src2d, dst2d):
    mesh = plsc.VectorSubcoreMesh(core_axis_name="c", subcore_axis_name="s",
                                  num_cores=NC, num_subcores=NS)
    fn = pl.kernel(
        _edge_body,
        out_type=[
            jax.ShapeDtypeStruct((OUT_ROWS, AW), _f32),
            jax.ShapeDtypeStruct((OUT_ROWS, AW), _f32),
            jax.ShapeDtypeStruct((OUT_ROWS,), _f32),
        ],
        mesh=mesh,
        compiler_params=pltpu.CompilerParams(needs_layout_passes=False,
                                             use_tc_tiling_on_sc=False),
        scratch_types=[
            pltpu.VMEM_SHARED((ACC_ROWS, AW), _f32),
            pltpu.VMEM_SHARED((ACC_ROWS,), _f32),
            pltpu.VMEM_SHARED((ADN,), _f32),
            pltpu.VMEM((BLK, TW), _f32),
            pltpu.VMEM((BLK, AW), _f32),
            pltpu.VMEM((BLK,), _f32),
            pltpu.VMEM((BLK,), _f32),
            pltpu.VMEM((ZROWS, AW), _f32),
            pltpu.VMEM((ZROWS,), _f32),
            pltpu.VMEM((SLAB, BLK), jnp.int32),
            pltpu.VMEM((SLAB, BLK), jnp.int32),
            pltpu.SemaphoreType.DMA,
        ],
    )
    return fn(ta, tb, ad, src2d, dst2d)


def _dense_c_body(oa_ref, ob_ref, dn_ref, ta_ref, tb_ref, xl_ref, wg2_ref,
                  asrc_ref, adst_ref, bg1_ref, t2a_ref, t2b_ref):
    ta = ta_ref[...]
    tb = tb_ref[...]
    as1 = ta[:, 16:17]
    ad1 = ta[:, 17:18]
    es = as1 + ad1
    ees = jnp.exp(jnp.maximum(es, 0.2 * es))
    g1 = jnp.concatenate([ta[:, :16], tb[:, :16]], axis=1)
    num = jnp.concatenate([oa_ref[...], ob_ref[...]], axis=1) + ees * g1
    den = dn_ref[...] + ees
    x1 = num / (den + 1e-16) + bg1_ref[...]
    x2in = jnp.concatenate([x1, xl_ref[...]], axis=1)
    g2 = jnp.dot(x2in, wg2_ref[...], preferred_element_type=_f32)
    as2 = jnp.sum(g2 * asrc_ref[...], axis=1, keepdims=True)
    ad2 = jnp.sum(g2 * adst_ref[...], axis=1, keepdims=True)
    padz = jnp.zeros((BN, TW - 18), _f32)
    t2a_ref[...] = jnp.concatenate([g2[:, :16], as2, ad2, padz], axis=1)
    t2b_ref[...] = jnp.concatenate([g2[:, 16:32], as2, ad2, padz], axis=1)


def _dense_c(oa, ob, dn, ta, tb, xl, wg2, asrc, adst, bg1):
    full = lambda shp: pl.BlockSpec(shp, lambda i: (0, 0))
    blk = lambda w: pl.BlockSpec((BN, w), lambda i: (i, 0))
    return pl.pallas_call(
        _dense_c_body,
        grid=(GRID,),
        in_specs=[blk(AW), blk(AW), blk(1), blk(TW), blk(TW), blk(IN_DIM),
                  full((OUT_CH + IN_DIM, OUT_CH)),
                  full((1, OUT_CH)), full((1, OUT_CH)), full((1, OUT_CH))],
        out_specs=[blk(TW), blk(TW)],
        out_shape=[jax.ShapeDtypeStruct((N, TW), _f32),
                   jax.ShapeDtypeStruct((N, TW), _f32)],
    )(oa, ob, dn, ta, tb, xl, wg2, asrc, adst, bg1)


def _dense_d_body(oa_ref, ob_ref, dn_ref, t2a_ref, t2b_ref, wlin_ref,
                  blin_ref, bg2_ref, out_ref):
    t2a = t2a_ref[...]
    t2b = t2b_ref[...]
    es = t2a[:, 16:17] + t2a[:, 17:18]
    ees = jnp.exp(jnp.maximum(es, 0.2 * es))
    g2 = jnp.concatenate([t2a[:, :16], t2b[:, :16]], axis=1)
    num = jnp.concatenate([oa_ref[...], ob_ref[...]], axis=1) + ees * g2
    den = dn_ref[...] + ees
    x2 = num / (den + 1e-16) + bg2_ref[...]
    logits = jnp.maximum(
        jnp.dot(x2, wlin_ref[...], preferred_element_type=_f32)
        + blin_ref[...], 0.0)
    m = jnp.max(logits, axis=1, keepdims=True)
    lg = logits - m
    out_ref[...] = lg - jnp.log(jnp.sum(jnp.exp(lg), axis=1, keepdims=True))


def _dense_d(oa, ob, dn, t2a, t2b, wlin, blin, bg2):
    full = lambda shp: pl.BlockSpec(shp, lambda i: (0, 0))
    blk = lambda w: pl.BlockSpec((BN, w), lambda i: (i, 0))
    return pl.pallas_call(
        _dense_d_body,
        grid=(GRID,),
        in_specs=[blk(AW), blk(AW), blk(1), blk(TW), blk(TW),
                  full((OUT_CH, NUM_CLASSES)), full((1, NUM_CLASSES)),
                  full((1, OUT_CH))],
        out_specs=blk(NUM_CLASSES),
        out_shape=jax.ShapeDtypeStruct((N, NUM_CLASSES), _f32),
    )(oa, ob, dn, t2a, t2b, wlin, blin, bg2)


@jax.jit
def _run(x, edge_index, W_ih, W_hh, b_ih, b_hh, W_att, W_fc, b_fc,
         W_g1, a_src1, a_dst1, b_g1, W_g2, a_src2, a_dst2, b_g2,
         W_lin, b_lin):
    x2d = x.reshape(N, LAG * IN_DIM)
    xlast = x[:, LAG - 1, :]
    pad = EP - E
    src2d = jnp.concatenate(
        [edge_index[0], jnp.zeros((pad,), jnp.int32)]).reshape(EP // BLK, BLK)
    dst2d = jnp.concatenate(
        [edge_index[1], jnp.full((pad,), N, jnp.int32)]).reshape(EP // BLK, BLK)
    wih = W_ih.T
    whh = W_hh.T
    b = (b_ih + b_hh)[None, :]
    ta, tb = _dense_a(x2d, wih, whh, b, W_att, W_fc, b_fc[None, :],
                      W_g1, a_src1[0][None, :], a_dst1[0][None, :])
    ad1 = jnp.concatenate([ta[:, 17], jnp.zeros((LANES,), _f32)])
    o1a, o1b, d1 = _edge_pass(ta, tb, ad1, src2d, dst2d)
    t2a, t2b = _dense_c(o1a[:N], o1b[:N], d1[:N, None], ta, tb, xlast, W_g2,
                        a_src2, a_dst2, b_g1[None, :])
    ad2 = jnp.concatenate([t2a[:, 17], jnp.zeros((LANES,), _f32)])
    o2a, o2b, d2 = _edge_pass(t2a, t2b, ad2, src2d, dst2d)
    return _dense_d(o2a[:N], o2b[:N], d2[:N, None], t2a, t2b,
                    W_lin, b_lin[None, :], b_g2[None, :])


def kernel(x, edge_index, W_ih, W_hh, b_ih, b_hh, W_att, W_fc, b_fc,
           W_g1, a_src1, a_dst1, b_g1, W_g2, a_src2, a_dst2, b_g2,
           W_lin, b_lin):
    return _run(x, edge_index, W_ih, W_hh, b_ih, b_hh, W_att, W_fc, b_fc,
                W_g1, a_src1, a_dst1, b_g1, W_g2, a_src2, a_dst2, b_g2,
                W_lin, b_lin)


# speedup vs baseline: 15.6337x; 1.1595x over previous
import jax
import jax.numpy as jnp
from jax import lax
from jax.experimental import pallas as pl
from jax.experimental.pallas import tpu as pltpu
from jax.experimental.pallas import tpu_sc as plsc

LAG = 10
IN_DIM = 12
HID = 64
OUT_CH = 32
NUM_CLASSES = 3
N = 100000
E = 1600000

NC = 2
NS = 16
LANES = 16

TW = 32
AW = 16
BLK = 128
SLAB = 4
NBLK = 800
NSLAB = NBLK // SLAB
EP = NC * 0 + NS * NBLK * BLK
ZROWS = 80
ACC_ROWS = 101120
ADN = N + LANES
WROWS = 6256
OUT_ROWS = NS * WROWS
BN = 2000
GRID = N // BN

_f32 = jnp.float32


def _dense_a_body(x_ref, wih_ref, whh_ref, b_ref, watt_ref, wfc_ref, bfc_ref,
                  wg1_ref, asrc_ref, adst_ref, ta_ref, tb_ref, hs_ref):
    xb = x_ref[...]
    wih = wih_ref[...]
    whh = whh_ref[...]
    bias = b_ref[...]
    h = jnp.zeros((BN, HID), _f32)
    c = jnp.zeros((BN, HID), _f32)
    for t in range(LAG):
        xt = xb[:, t * IN_DIM:(t + 1) * IN_DIM]
        gates = (jnp.dot(xt, wih, preferred_element_type=_f32)
                 + jnp.dot(h, whh, preferred_element_type=_f32) + bias)
        ig = jax.nn.sigmoid(gates[:, 0:HID])
        fg = jax.nn.sigmoid(gates[:, HID:2 * HID])
        gg = jnp.tanh(gates[:, 2 * HID:3 * HID])
        og = jax.nn.sigmoid(gates[:, 3 * HID:4 * HID])
        c = fg * c + ig * gg
        h = og * jnp.tanh(c)
        hs_ref[:, t * HID:(t + 1) * HID] = h
    hs = hs_ref[...]
    watt = watt_ref[...]
    cols = []
    for t in range(LAG):
        ht = hs[:, t * HID:(t + 1) * HID]
        cols.append(jnp.sum(ht * watt[t:t + 1, :], axis=1, keepdims=True))
    sc = jnp.concatenate(cols, axis=1)
    m = jnp.max(sc, axis=1, keepdims=True)
    ex = jnp.exp(sc - m)
    att = ex / jnp.sum(ex, axis=1, keepdims=True)
    att_ht = jnp.zeros((BN, HID), _f32)
    for t in range(LAG):
        att_ht = att_ht + att[:, t:t + 1] * hs[:, t * HID:(t + 1) * HID]
    hfc = jnp.maximum(
        jnp.dot(att_ht, wfc_ref[...], preferred_element_type=_f32)
        + bfc_ref[...], 0.0)
    g1 = jnp.dot(hfc, wg1_ref[...], preferred_element_type=_f32)
    as1 = jnp.sum(g1 * asrc_ref[...], axis=1, keepdims=True)
    ad1 = jnp.sum(g1 * adst_ref[...], axis=1, keepdims=True)
    padz = jnp.zeros((BN, TW - 18), _f32)
    ta_ref[...] = jnp.concatenate([g1[:, :16], as1, ad1, padz], axis=1)
    tb_ref[...] = jnp.concatenate([g1[:, 16:32], as1, ad1, padz], axis=1)


def _dense_a(x2d, wih, whh, b, watt, wfc, bfc, wg1, asrc, adst):
    full = lambda shp: pl.BlockSpec(shp, lambda i: (0, 0))
    return pl.pallas_call(
        _dense_a_body,
        grid=(GRID,),
        in_specs=[
            pl.BlockSpec((BN, LAG * IN_DIM), lambda i: (i, 0)),
            full((IN_DIM, 4 * HID)),
            full((HID, 4 * HID)),
            full((1, 4 * HID)),
            full((LAG, HID)),
            full((HID, HID)),
            full((1, HID)),
            full((HID, OUT_CH)),
            full((1, OUT_CH)),
            full((1, OUT_CH)),
        ],
        out_specs=[
            pl.BlockSpec((BN, TW), lambda i: (i, 0)),
            pl.BlockSpec((BN, TW), lambda i: (i, 0)),
        ],
        out_shape=[
            jax.ShapeDtypeStruct((N, TW), _f32),
            jax.ShapeDtypeStruct((N, TW), _f32),
        ],
        scratch_shapes=[pltpu.VMEM((BN, LAG * HID), _f32)],
    )(x2d, wih, whh, b, watt, wfc, bfc, wg1, asrc, adst)


def _edge_body(ta_hbm, tb_hbm, ad_hbm, src_hbm, dst_hbm,
               outa_hbm, outb_hbm, den_hbm,
               acc, den_acc, ad_s, rows0, rows1, out_v, ee_v, adv, zrow, zden,
               sidx, didx, sem0, sem1):
    core = lax.axis_index("c")
    tile = lax.axis_index("s")

    zeros16 = jnp.zeros((LANES,), _f32)

    def _zr(r, _):
        zrow[r, 0:LANES] = zeros16
        return 0

    lax.fori_loop(0, ZROWS, _zr, 0)
    for g in range(ZROWS // LANES):
        zden[pl.ds(g * LANES, LANES)] = zeros16

    rows_per_tile = ACC_ROWS // NS
    zbase = tile * rows_per_tile

    def _zacc(z, _):
        pltpu.sync_copy(zrow, acc.at[pl.ds(zbase + z * ZROWS, ZROWS)])
        pltpu.sync_copy(zden, den_acc.at[pl.ds(zbase + z * ZROWS, ZROWS)])
        return 0

    lax.fori_loop(0, rows_per_tile // ZROWS, _zacc, 0)

    @pl.when(tile == 0)
    def _stage_ad():
        pltpu.sync_copy(ad_hbm, ad_s)

    plsc.subcore_barrier()

    cols16 = {}

    def _c16(c):
        if c not in cols16:
            cols16[c] = jnp.full((LANES,), c, jnp.int32)
        return cols16[c]

    bufs = (rows0, rows1)
    sems = (sem0, sem1)

    def _issue(j):
        b, s = bufs[j % 2], sems[j % 2]

        def _ia():
            pltpu.async_copy(ta_hbm.at[sidx.at[j]], b, s)

        def _ib():
            pltpu.async_copy(tb_hbm.at[sidx.at[j]], b, s)

        pl.when(core == 0)(_ia)
        pl.when(core == 1)(_ib)

    def _wait(j):
        b, s = bufs[j % 2], sems[j % 2]
        pl.when(core == 0)(
            lambda: pltpu.make_async_copy(ta_hbm.at[sidx.at[j]], b, s).wait())
        pl.when(core == 1)(
            lambda: pltpu.make_async_copy(tb_hbm.at[sidx.at[j]], b, s).wait())

    def _compute(j):
        rows = bufs[j % 2]
        pltpu.sync_copy(ad_s.at[didx.at[j]], adv)
        for g in range(BLK // LANES):
            eids = lax.iota(jnp.int32, LANES) + (g * LANES)
            as16 = plsc.load_gather(rows, [eids, _c16(16)])
            ad16 = adv[pl.ds(g * LANES, LANES)]
            e = as16 + ad16
            ee = jnp.exp(jnp.maximum(e, 0.2 * e))
            ee_v[pl.ds(g * LANES, LANES)] = ee
            for col in range(AW):
                v = plsc.load_gather(rows, [eids, _c16(col)])
                plsc.store_scatter(out_v, [eids, _c16(col)], v * ee)
        pltpu.sync_copy(out_v, acc.at[didx.at[j]], add=True)
        pl.when(core == 0)(
            lambda: pltpu.sync_copy(ee_v, den_acc.at[didx.at[j]], add=True))

    def _slab(sl, _):
        srow = tile * NBLK + sl * SLAB
        pltpu.sync_copy(src_hbm.at[pl.ds(srow, SLAB)], sidx)
        pltpu.sync_copy(dst_hbm.at[pl.ds(srow, SLAB)], didx)
        _issue(0)
        for j in range(SLAB):
            if j + 1 < SLAB:
                _issue(j + 1)
            _wait(j)
            _compute(j)
        return 0

    lax.fori_loop(0, NSLAB, _slab, 0)
    plsc.subcore_barrier()

    wbase = tile * WROWS
    pl.when(core == 0)(lambda: pltpu.sync_copy(
        acc.at[pl.ds(wbase, WROWS)], outa_hbm.at[pl.ds(wbase, WROWS)]))
    pl.when(core == 0)(lambda: pltpu.sync_copy(
        den_acc.at[pl.ds(wbase, WROWS)], den_hbm.at[pl.ds(wbase, WROWS)]))
    pl.when(core == 1)(lambda: pltpu.sync_copy(
        acc.at[pl.ds(wbase, WROWS)], outb_hbm.at[pl.ds(wbase, WROWS)]))


def _edge_pass(ta, tb, ad, src2d, dst2d):
    mesh = plsc.VectorSubcoreMesh(core_axis_name="c", subcore_axis_name="s",
                                  num_cores=NC, num_subcores=NS)
    fn = pl.kernel(
        _edge_body,
        out_type=[
            jax.ShapeDtypeStruct((OUT_ROWS, AW), _f32),
            jax.ShapeDtypeStruct((OUT_ROWS, AW), _f32),
            jax.ShapeDtypeStruct((OUT_ROWS,), _f32),
        ],
        mesh=mesh,
        compiler_params=pltpu.CompilerParams(needs_layout_passes=False,
                                             use_tc_tiling_on_sc=False),
        scratch_types=[
            pltpu.VMEM_SHARED((ACC_ROWS, AW), _f32),
            pltpu.VMEM_SHARED((ACC_ROWS,), _f32),
            pltpu.VMEM_SHARED((ADN,), _f32),
            pltpu.VMEM((BLK, TW), _f32),
            pltpu.VMEM((BLK, TW), _f32),
            pltpu.VMEM((BLK, AW), _f32),
            pltpu.VMEM((BLK,), _f32),
            pltpu.VMEM((BLK,), _f32),
            pltpu.VMEM((ZROWS, AW), _f32),
            pltpu.VMEM((ZROWS,), _f32),
            pltpu.VMEM((SLAB, BLK), jnp.int32),
            pltpu.VMEM((SLAB, BLK), jnp.int32),
            pltpu.SemaphoreType.DMA,
            pltpu.SemaphoreType.DMA,
        ],
    )
    return fn(ta, tb, ad, src2d, dst2d)


def _dense_c_body(oa_ref, ob_ref, dn_ref, ta_ref, tb_ref, xl_ref, wg2_ref,
                  asrc_ref, adst_ref, bg1_ref, t2a_ref, t2b_ref):
    ta = ta_ref[...]
    tb = tb_ref[...]
    as1 = ta[:, 16:17]
    ad1 = ta[:, 17:18]
    es = as1 + ad1
    ees = jnp.exp(jnp.maximum(es, 0.2 * es))
    g1 = jnp.concatenate([ta[:, :16], tb[:, :16]], axis=1)
    num = jnp.concatenate([oa_ref[...], ob_ref[...]], axis=1) + ees * g1
    den = dn_ref[...] + ees
    x1 = num / (den + 1e-16) + bg1_ref[...]
    x2in = jnp.concatenate([x1, xl_ref[...]], axis=1)
    g2 = jnp.dot(x2in, wg2_ref[...], preferred_element_type=_f32)
    as2 = jnp.sum(g2 * asrc_ref[...], axis=1, keepdims=True)
    ad2 = jnp.sum(g2 * adst_ref[...], axis=1, keepdims=True)
    padz = jnp.zeros((BN, TW - 18), _f32)
    t2a_ref[...] = jnp.concatenate([g2[:, :16], as2, ad2, padz], axis=1)
    t2b_ref[...] = jnp.concatenate([g2[:, 16:32], as2, ad2, padz], axis=1)


def _dense_c(oa, ob, dn, ta, tb, xl, wg2, asrc, adst, bg1):
    full = lambda shp: pl.BlockSpec(shp, lambda i: (0, 0))
    blk = lambda w: pl.BlockSpec((BN, w), lambda i: (i, 0))
    return pl.pallas_call(
        _dense_c_body,
        grid=(GRID,),
        in_specs=[blk(AW), blk(AW), blk(1), blk(TW), blk(TW), blk(IN_DIM),
                  full((OUT_CH + IN_DIM, OUT_CH)),
                  full((1, OUT_CH)), full((1, OUT_CH)), full((1, OUT_CH))],
        out_specs=[blk(TW), blk(TW)],
        out_shape=[jax.ShapeDtypeStruct((N, TW), _f32),
                   jax.ShapeDtypeStruct((N, TW), _f32)],
    )(oa, ob, dn, ta, tb, xl, wg2, asrc, adst, bg1)


def _dense_d_body(oa_ref, ob_ref, dn_ref, t2a_ref, t2b_ref, wlin_ref,
                  blin_ref, bg2_ref, out_ref):
    t2a = t2a_ref[...]
    t2b = t2b_ref[...]
    es = t2a[:, 16:17] + t2a[:, 17:18]
    ees = jnp.exp(jnp.maximum(es, 0.2 * es))
    g2 = jnp.concatenate([t2a[:, :16], t2b[:, :16]], axis=1)
    num = jnp.concatenate([oa_ref[...], ob_ref[...]], axis=1) + ees * g2
    den = dn_ref[...] + ees
    x2 = num / (den + 1e-16) + bg2_ref[...]
    logits = jnp.maximum(
        jnp.dot(x2, wlin_ref[...], preferred_element_type=_f32)
        + blin_ref[...], 0.0)
    m = jnp.max(logits, axis=1, keepdims=True)
    lg = logits - m
    out_ref[...] = lg - jnp.log(jnp.sum(jnp.exp(lg), axis=1, keepdims=True))


def _dense_d(oa, ob, dn, t2a, t2b, wlin, blin, bg2):
    full = lambda shp: pl.BlockSpec(shp, lambda i: (0, 0))
    blk = lambda w: pl.BlockSpec((BN, w), lambda i: (i, 0))
    return pl.pallas_call(
        _dense_d_body,
        grid=(GRID,),
        in_specs=[blk(AW), blk(AW), blk(1), blk(TW), blk(TW),
                  full((OUT_CH, NUM_CLASSES)), full((1, NUM_CLASSES)),
                  full((1, OUT_CH))],
        out_specs=blk(NUM_CLASSES),
        out_shape=jax.ShapeDtypeStruct((N, NUM_CLASSES), _f32),
    )(oa, ob, dn, t2a, t2b, wlin, blin, bg2)


@jax.jit
def _run(x, edge_index, W_ih, W_hh, b_ih, b_hh, W_att, W_fc, b_fc,
         W_g1, a_src1, a_dst1, b_g1, W_g2, a_src2, a_dst2, b_g2,
         W_lin, b_lin):
    x2d = x.reshape(N, LAG * IN_DIM)
    xlast = x[:, LAG - 1, :]
    pad = EP - E
    src2d = jnp.concatenate(
        [edge_index[0], jnp.zeros((pad,), jnp.int32)]).reshape(EP // BLK, BLK)
    dst2d = jnp.concatenate(
        [edge_index[1], jnp.full((pad,), N, jnp.int32)]).reshape(EP // BLK, BLK)
    wih = W_ih.T
    whh = W_hh.T
    b = (b_ih + b_hh)[None, :]
    ta, tb = _dense_a(x2d, wih, whh, b, W_att, W_fc, b_fc[None, :],
                      W_g1, a_src1[0][None, :], a_dst1[0][None, :])
    ad1 = jnp.concatenate([ta[:, 17], jnp.zeros((LANES,), _f32)])
    o1a, o1b, d1 = _edge_pass(ta, tb, ad1, src2d, dst2d)
    t2a, t2b = _dense_c(o1a[:N], o1b[:N], d1[:N, None], ta, tb, xlast, W_g2,
                        a_src2, a_dst2, b_g1[None, :])
    ad2 = jnp.concatenate([t2a[:, 17], jnp.zeros((LANES,), _f32)])
    o2a, o2b, d2 = _edge_pass(t2a, t2b, ad2, src2d, dst2d)
    return _dense_d(o2a[:N], o2b[:N], d2[:N, None], t2a, t2b,
                    W_lin, b_lin[None, :], b_g2[None, :])


def kernel(x, edge_index, W_ih, W_hh, b_ih, b_hh, W_att, W_fc, b_fc,
           W_g1, a_src1, a_dst1, b_g1, W_g2, a_src2, a_dst2, b_g2,
           W_lin, b_lin):
    return _run(x, edge_index, W_ih, W_hh, b_ih, b_hh, W_att, W_fc, b_fc,
                W_g1, a_src1, a_dst1, b_g1, W_g2, a_src2, a_dst2, b_g2,
                W_lin, b_lin)


# speedup vs baseline: 16.1452x; 1.0327x over previous
import jax
import jax.numpy as jnp
from jax import lax
from jax.experimental import pallas as pl
from jax.experimental.pallas import tpu as pltpu
from jax.experimental.pallas import tpu_sc as plsc

LAG = 10
IN_DIM = 12
HID = 64
OUT_CH = 32
NUM_CLASSES = 3
N = 100000
E = 1600000

NC = 2
NS = 16
LANES = 16

TW = 32
AW = 16
BLK = 128
SLAB = 5
NBLK = 800
NSLAB = NBLK // SLAB
EP = NC * 0 + NS * NBLK * BLK
ZROWS = 80
ACC_ROWS = 101120
ADN = N + LANES
WROWS = 6256
OUT_ROWS = NS * WROWS
BN = 2000
GRID = N // BN

_f32 = jnp.float32


def _dense_a_body(x_ref, wih_ref, whh_ref, b_ref, watt_ref, wfc_ref, bfc_ref,
                  wg1_ref, asrc_ref, adst_ref, ta_ref, tb_ref, hs_ref):
    xb = x_ref[...]
    wih = wih_ref[...]
    whh = whh_ref[...]
    bias = b_ref[...]
    h = jnp.zeros((BN, HID), _f32)
    c = jnp.zeros((BN, HID), _f32)
    for t in range(LAG):
        xt = xb[:, t * IN_DIM:(t + 1) * IN_DIM]
        gates = (jnp.dot(xt, wih, preferred_element_type=_f32)
                 + jnp.dot(h, whh, preferred_element_type=_f32) + bias)
        ig = jax.nn.sigmoid(gates[:, 0:HID])
        fg = jax.nn.sigmoid(gates[:, HID:2 * HID])
        gg = jnp.tanh(gates[:, 2 * HID:3 * HID])
        og = jax.nn.sigmoid(gates[:, 3 * HID:4 * HID])
        c = fg * c + ig * gg
        h = og * jnp.tanh(c)
        hs_ref[:, t * HID:(t + 1) * HID] = h
    hs = hs_ref[...]
    watt = watt_ref[...]
    cols = []
    for t in range(LAG):
        ht = hs[:, t * HID:(t + 1) * HID]
        cols.append(jnp.sum(ht * watt[t:t + 1, :], axis=1, keepdims=True))
    sc = jnp.concatenate(cols, axis=1)
    m = jnp.max(sc, axis=1, keepdims=True)
    ex = jnp.exp(sc - m)
    att = ex / jnp.sum(ex, axis=1, keepdims=True)
    att_ht = jnp.zeros((BN, HID), _f32)
    for t in range(LAG):
        att_ht = att_ht + att[:, t:t + 1] * hs[:, t * HID:(t + 1) * HID]
    hfc = jnp.maximum(
        jnp.dot(att_ht, wfc_ref[...], preferred_element_type=_f32)
        + bfc_ref[...], 0.0)
    g1 = jnp.dot(hfc, wg1_ref[...], preferred_element_type=_f32)
    as1 = jnp.sum(g1 * asrc_ref[...], axis=1, keepdims=True)
    ad1 = jnp.sum(g1 * adst_ref[...], axis=1, keepdims=True)
    padz = jnp.zeros((BN, TW - 18), _f32)
    ta_ref[...] = jnp.concatenate([g1[:, :16], as1, ad1, padz], axis=1)
    tb_ref[...] = jnp.concatenate([g1[:, 16:32], as1, ad1, padz], axis=1)


def _dense_a(x2d, wih, whh, b, watt, wfc, bfc, wg1, asrc, adst):
    full = lambda shp: pl.BlockSpec(shp, lambda i: (0, 0))
    return pl.pallas_call(
        _dense_a_body,
        grid=(GRID,),
        in_specs=[
            pl.BlockSpec((BN, LAG * IN_DIM), lambda i: (i, 0)),
            full((IN_DIM, 4 * HID)),
            full((HID, 4 * HID)),
            full((1, 4 * HID)),
            full((LAG, HID)),
            full((HID, HID)),
            full((1, HID)),
            full((HID, OUT_CH)),
            full((1, OUT_CH)),
            full((1, OUT_CH)),
        ],
        out_specs=[
            pl.BlockSpec((BN, TW), lambda i: (i, 0)),
            pl.BlockSpec((BN, TW), lambda i: (i, 0)),
        ],
        out_shape=[
            jax.ShapeDtypeStruct((N, TW), _f32),
            jax.ShapeDtypeStruct((N, TW), _f32),
        ],
        scratch_shapes=[pltpu.VMEM((BN, LAG * HID), _f32)],
    )(x2d, wih, whh, b, watt, wfc, bfc, wg1, asrc, adst)


def _edge_body(ta_hbm, tb_hbm, ad_hbm, src_hbm, dst_hbm,
               outa_hbm, outb_hbm, den_hbm,
               acc, den_acc, ad_s, rows0, rows1, out_v, ee_v, adv,
               zrow, zden, sidx, didx, semr0, semr1):
    core = lax.axis_index("c")
    tile = lax.axis_index("s")

    zeros16 = jnp.zeros((LANES,), _f32)

    def _zr(r, _):
        zrow[r, 0:LANES] = zeros16
        return 0

    lax.fori_loop(0, ZROWS, _zr, 0)
    for g in range(ZROWS // LANES):
        zden[pl.ds(g * LANES, LANES)] = zeros16

    rows_per_tile = ACC_ROWS // NS
    zbase = tile * rows_per_tile

    def _zacc(z, _):
        pltpu.sync_copy(zrow, acc.at[pl.ds(zbase + z * ZROWS, ZROWS)])
        pltpu.sync_copy(zden, den_acc.at[pl.ds(zbase + z * ZROWS, ZROWS)])
        return 0

    lax.fori_loop(0, rows_per_tile // ZROWS, _zacc, 0)

    @pl.when(tile == 0)
    def _stage_ad():
        pltpu.sync_copy(ad_hbm, ad_s)

    plsc.subcore_barrier()

    cols16 = {}

    def _c16(c):
        if c not in cols16:
            cols16[c] = jnp.full((LANES,), c, jnp.int32)
        return cols16[c]

    rbufs = (rows0, rows1)
    rsems = (semr0, semr1)

    def _issue(j):
        b, s = rbufs[j % 2], rsems[j % 2]

        def _ia():
            pltpu.async_copy(ta_hbm.at[sidx.at[j]], b, s)

        def _ib():
            pltpu.async_copy(tb_hbm.at[sidx.at[j]], b, s)

        pl.when(core == 0)(_ia)
        pl.when(core == 1)(_ib)

    def _wait(j):
        b, s = rbufs[j % 2], rsems[j % 2]
        pl.when(core == 0)(
            lambda: pltpu.make_async_copy(ta_hbm.at[sidx.at[j]], b, s).wait())
        pl.when(core == 1)(
            lambda: pltpu.make_async_copy(tb_hbm.at[sidx.at[j]], b, s).wait())

    def _compute(j):
        rows = rbufs[j % 2]
        pltpu.sync_copy(ad_s.at[didx.at[j]], adv)
        for g in range(BLK // LANES):
            eids = lax.iota(jnp.int32, LANES) + (g * LANES)
            as16 = plsc.load_gather(rows, [eids, _c16(16)])
            ad16 = adv[pl.ds(g * LANES, LANES)]
            e = as16 + ad16
            ee = jnp.exp(jnp.maximum(e, 0.2 * e))
            ee_v[pl.ds(g * LANES, LANES)] = ee
            for col in range(AW):
                v = plsc.load_gather(rows, [eids, _c16(col)])
                plsc.store_scatter(out_v, [eids, _c16(col)], v * ee)
        pltpu.sync_copy(out_v, acc.at[didx.at[j]], add=True)
        pl.when(core == 0)(
            lambda: pltpu.sync_copy(ee_v, den_acc.at[didx.at[j]], add=True))

    def _slab(sl, _):
        srow = tile * NBLK + sl * SLAB
        pltpu.sync_copy(src_hbm.at[pl.ds(srow, SLAB)], sidx)
        pltpu.sync_copy(dst_hbm.at[pl.ds(srow, SLAB)], didx)
        _issue(0)
        for j in range(SLAB):
            if j + 1 < SLAB:
                _issue(j + 1)
            _wait(j)
            _compute(j)
        return 0

    lax.fori_loop(0, NSLAB, _slab, 0)
    plsc.subcore_barrier()

    wbase = tile * WROWS
    pl.when(core == 0)(lambda: pltpu.sync_copy(
        acc.at[pl.ds(wbase, WROWS)], outa_hbm.at[pl.ds(wbase, WROWS)]))
    pl.when(core == 0)(lambda: pltpu.sync_copy(
        den_acc.at[pl.ds(wbase, WROWS)], den_hbm.at[pl.ds(wbase, WROWS)]))
    pl.when(core == 1)(lambda: pltpu.sync_copy(
        acc.at[pl.ds(wbase, WROWS)], outb_hbm.at[pl.ds(wbase, WROWS)]))


def _edge_pass(ta, tb, ad, src2d, dst2d):
    mesh = plsc.VectorSubcoreMesh(core_axis_name="c", subcore_axis_name="s",
                                  num_cores=NC, num_subcores=NS)
    fn = pl.kernel(
        _edge_body,
        out_type=[
            jax.ShapeDtypeStruct((OUT_ROWS, AW), _f32),
            jax.ShapeDtypeStruct((OUT_ROWS, AW), _f32),
            jax.ShapeDtypeStruct((OUT_ROWS,), _f32),
        ],
        mesh=mesh,
        compiler_params=pltpu.CompilerParams(needs_layout_passes=False,
                                             use_tc_tiling_on_sc=False),
        scratch_types=[
            pltpu.VMEM_SHARED((ACC_ROWS, AW), _f32),
            pltpu.VMEM_SHARED((ACC_ROWS,), _f32),
            pltpu.VMEM_SHARED((ADN,), _f32),
            pltpu.VMEM((BLK, TW), _f32),
            pltpu.VMEM((BLK, TW), _f32),
            pltpu.VMEM((BLK, AW), _f32),
            pltpu.VMEM((BLK,), _f32),
            pltpu.VMEM((BLK,), _f32),
            pltpu.VMEM((ZROWS, AW), _f32),
            pltpu.VMEM((ZROWS,), _f32),
            pltpu.VMEM((SLAB, BLK), jnp.int32),
            pltpu.VMEM((SLAB, BLK), jnp.int32),
        ] + [pltpu.SemaphoreType.DMA] * 2,
    )
    return fn(ta, tb, ad, src2d, dst2d)


def _dense_c_body(oa_ref, ob_ref, dn_ref, ta_ref, tb_ref, xl_ref, wg2_ref,
                  asrc_ref, adst_ref, bg1_ref, t2a_ref, t2b_ref):
    ta = ta_ref[...]
    tb = tb_ref[...]
    as1 = ta[:, 16:17]
    ad1 = ta[:, 17:18]
    es = as1 + ad1
    ees = jnp.exp(jnp.maximum(es, 0.2 * es))
    g1 = jnp.concatenate([ta[:, :16], tb[:, :16]], axis=1)
    num = jnp.concatenate([oa_ref[...], ob_ref[...]], axis=1) + ees * g1
    den = dn_ref[...] + ees
    x1 = num / (den + 1e-16) + bg1_ref[...]
    x2in = jnp.concatenate([x1, xl_ref[...]], axis=1)
    g2 = jnp.dot(x2in, wg2_ref[...], preferred_element_type=_f32)
    as2 = jnp.sum(g2 * asrc_ref[...], axis=1, keepdims=True)
    ad2 = jnp.sum(g2 * adst_ref[...], axis=1, keepdims=True)
    padz = jnp.zeros((BN, TW - 18), _f32)
    t2a_ref[...] = jnp.concatenate([g2[:, :16], as2, ad2, padz], axis=1)
    t2b_ref[...] = jnp.concatenate([g2[:, 16:32], as2, ad2, padz], axis=1)


def _dense_c(oa, ob, dn, ta, tb, xl, wg2, asrc, adst, bg1):
    full = lambda shp: pl.BlockSpec(shp, lambda i: (0, 0))
    blk = lambda w: pl.BlockSpec((BN, w), lambda i: (i, 0))
    return pl.pallas_call(
        _dense_c_body,
        grid=(GRID,),
        in_specs=[blk(AW), blk(AW), blk(1), blk(TW), blk(TW), blk(IN_DIM),
                  full((OUT_CH + IN_DIM, OUT_CH)),
                  full((1, OUT_CH)), full((1, OUT_CH)), full((1, OUT_CH))],
        out_specs=[blk(TW), blk(TW)],
        out_shape=[jax.ShapeDtypeStruct((N, TW), _f32),
                   jax.ShapeDtypeStruct((N, TW), _f32)],
    )(oa, ob, dn, ta, tb, xl, wg2, asrc, adst, bg1)


def _dense_d_body(oa_ref, ob_ref, dn_ref, t2a_ref, t2b_ref, wlin_ref,
                  blin_ref, bg2_ref, out_ref):
    t2a = t2a_ref[...]
    t2b = t2b_ref[...]
    es = t2a[:, 16:17] + t2a[:, 17:18]
    ees = jnp.exp(jnp.maximum(es, 0.2 * es))
    g2 = jnp.concatenate([t2a[:, :16], t2b[:, :16]], axis=1)
    num = jnp.concatenate([oa_ref[...], ob_ref[...]], axis=1) + ees * g2
    den = dn_ref[...] + ees
    x2 = num / (den + 1e-16) + bg2_ref[...]
    logits = jnp.maximum(
        jnp.dot(x2, wlin_ref[...], preferred_element_type=_f32)
        + blin_ref[...], 0.0)
    m = jnp.max(logits, axis=1, keepdims=True)
    lg = logits - m
    out_ref[...] = lg - jnp.log(jnp.sum(jnp.exp(lg), axis=1, keepdims=True))


def _dense_d(oa, ob, dn, t2a, t2b, wlin, blin, bg2):
    full = lambda shp: pl.BlockSpec(shp, lambda i: (0, 0))
    blk = lambda w: pl.BlockSpec((BN, w), lambda i: (i, 0))
    return pl.pallas_call(
        _dense_d_body,
        grid=(GRID,),
        in_specs=[blk(AW), blk(AW), blk(1), blk(TW), blk(TW),
                  full((OUT_CH, NUM_CLASSES)), full((1, NUM_CLASSES)),
                  full((1, OUT_CH))],
        out_specs=blk(NUM_CLASSES),
        out_shape=jax.ShapeDtypeStruct((N, NUM_CLASSES), _f32),
    )(oa, ob, dn, t2a, t2b, wlin, blin, bg2)


@jax.jit
def _run(x, edge_index, W_ih, W_hh, b_ih, b_hh, W_att, W_fc, b_fc,
         W_g1, a_src1, a_dst1, b_g1, W_g2, a_src2, a_dst2, b_g2,
         W_lin, b_lin):
    x2d = x.reshape(N, LAG * IN_DIM)
    xlast = x[:, LAG - 1, :]
    pad = EP - E
    src2d = jnp.concatenate(
        [edge_index[0], jnp.zeros((pad,), jnp.int32)]).reshape(EP // BLK, BLK)
    dst2d = jnp.concatenate(
        [edge_index[1], jnp.full((pad,), N, jnp.int32)]).reshape(EP // BLK, BLK)
    wih = W_ih.T
    whh = W_hh.T
    b = (b_ih + b_hh)[None, :]
    ta, tb = _dense_a(x2d, wih, whh, b, W_att, W_fc, b_fc[None, :],
                      W_g1, a_src1[0][None, :], a_dst1[0][None, :])
    ad1 = jnp.concatenate([ta[:, 17], jnp.zeros((LANES,), _f32)])
    o1a, o1b, d1 = _edge_pass(ta, tb, ad1, src2d, dst2d)
    t2a, t2b = _dense_c(o1a[:N], o1b[:N], d1[:N, None], ta, tb, xlast, W_g2,
                        a_src2, a_dst2, b_g1[None, :])
    ad2 = jnp.concatenate([t2a[:, 17], jnp.zeros((LANES,), _f32)])
    o2a, o2b, d2 = _edge_pass(t2a, t2b, ad2, src2d, dst2d)
    return _dense_d(o2a[:N], o2b[:N], d2[:N, None], t2a, t2b,
                    W_lin, b_lin[None, :], b_g2[None, :])


def kernel(x, edge_index, W_ih, W_hh, b_ih, b_hh, W_att, W_fc, b_fc,
           W_g1, a_src1, a_dst1, b_g1, W_g2, a_src2, a_dst2, b_g2,
           W_lin, b_lin):
    return _run(x, edge_index, W_ih, W_hh, b_ih, b_hh, W_att, W_fc, b_fc,
                W_g1, a_src1, a_dst1, b_g1, W_g2, a_src2, a_dst2, b_g2,
                W_lin, b_lin)
